# Initial kernel scaffold; baseline (speedup 1.0000x reference)
import jax
import jax.numpy as jnp
from jax.experimental import pallas as pl


def kernel(user_emb, item_emb, W0, W1, u_row, u_col, u_val, i_row, i_col, i_val):
    raise NotImplementedError("write your pallas kernel here")


# speedup vs baseline: 1.9898x; 1.9898x over previous
import functools

import jax
import jax.numpy as jnp
from jax import lax
from jax.experimental import pallas as pl
from jax.experimental.pallas import tpu as pltpu
from jax.experimental.pallas import tpu_sc as plsc

N = 50000
NP = 50048
D = 64
H = 32
E = 800000
C = 80
NCHUNK = E // C
NTILE = 16
CPT = NCHUNK // NTILE
RPT = NP // NTILE
ZCH = 782
RBLK = 3128


def _spmm_body(x2, col1, row1, val1, y, colv, rowv, valv, gbuf, zbuf, acc, sem):
    c = lax.axis_index("c")
    s = lax.axis_index("s")

    def zrow(i, _):
        zbuf[i, pl.ds(0, 16)] = jnp.zeros((16,), jnp.float32)
        zbuf[i, pl.ds(16, 16)] = jnp.zeros((16,), jnp.float32)
        return 0
    lax.fori_loop(0, ZCH, zrow, 0)
    for k in range(RPT // ZCH):
        pltpu.sync_copy(zbuf, acc.at[pl.ds(s * RPT + k * ZCH, ZCH)])
    plsc.subcore_barrier()

    base_adj = c * NP

    def chunk(j, _):
        e0 = (s * CPT + j) * C
        pltpu.sync_copy(col1.at[pl.ds(e0, C)], colv)
        pltpu.sync_copy(row1.at[pl.ds(e0, C)], rowv)
        pltpu.sync_copy(val1.at[pl.ds(e0, C)], valv)
        for i in range(C // 16):
            colv[pl.ds(i * 16, 16)] = colv[pl.ds(i * 16, 16)] + base_adj
        pltpu.async_copy(x2.at[colv], gbuf, sem).wait()
        for b in range(C // 16):
            vvec = valv[pl.ds(b * 16, 16)]
            for i in range(16):
                e = b * 16 + i
                v = vvec[i]
                gbuf[e, pl.ds(0, 16)] = gbuf[e, pl.ds(0, 16)] * v
                gbuf[e, pl.ds(16, 16)] = gbuf[e, pl.ds(16, 16)] * v
        pltpu.sync_copy(gbuf, acc.at[rowv], add=True)
        return 0

    lax.fori_loop(0, CPT, chunk, 0)
    plsc.subcore_barrier()

    pltpu.sync_copy(acc.at[pl.ds(s * RPT, RPT)], y.at[c, pl.ds(s * RPT, RPT)])


def _make_spmm():
    mesh = plsc.VectorSubcoreMesh(core_axis_name="c", subcore_axis_name="s")
    return functools.partial(
        pl.kernel,
        mesh=mesh,
        compiler_params=pltpu.CompilerParams(use_tc_tiling_on_sc=False),
        out_type=jax.ShapeDtypeStruct((2, NP, H), jnp.float32),
        scratch_types=[
            pltpu.VMEM((C,), jnp.int32),
            pltpu.VMEM((C,), jnp.int32),
            pltpu.VMEM((C,), jnp.float32),
            pltpu.VMEM((C, H), jnp.float32),
            pltpu.VMEM((ZCH, H), jnp.float32),
            pltpu.VMEM_SHARED((NP, H), jnp.float32),
            pltpu.SemaphoreType.DMA,
        ],
    )(_spmm_body)


_spmm = _make_spmm()


def _dense_body(y0, y1, w, z, h0, h1):
    nu = jnp.concatenate([y0[...], y1[...]], axis=1)
    zz = jnp.dot(nu, w[...], preferred_element_type=jnp.float32) + nu
    zz = jnp.where(zz >= 0, zz, 0.01 * zz)
    nrm = jnp.sqrt(jnp.sum(zz * zz, axis=1, keepdims=True))
    zz = zz / jnp.maximum(nrm, 1e-12)
    z[...] = zz
    h0[...] = zz[:, :H]
    h1[...] = zz[:, H:]


def _dense(y, w):
    grid = NP // RBLK
    return pl.pallas_call(
        _dense_body,
        grid=(grid,),
        in_specs=[
            pl.BlockSpec((RBLK, H), lambda i: (i, 0)),
            pl.BlockSpec((RBLK, H), lambda i: (i, 0)),
            pl.BlockSpec((D, D), lambda i: (0, 0)),
        ],
        out_specs=[
            pl.BlockSpec((RBLK, D), lambda i: (i, 0)),
            pl.BlockSpec((RBLK, H), lambda i: (i, 0)),
            pl.BlockSpec((RBLK, H), lambda i: (i, 0)),
        ],
        out_shape=[
            jax.ShapeDtypeStruct((NP, D), jnp.float32),
            jax.ShapeDtypeStruct((NP, H), jnp.float32),
            jax.ShapeDtypeStruct((NP, H), jnp.float32),
        ],
    )(y[0], y[1], w)


def kernel(user_emb, item_emb, W0, W1, u_row, u_col, u_val, i_row, i_col, i_val):
    pad = jnp.zeros((NP - N, H), jnp.float32)
    xu = jnp.concatenate([user_emb[:, :H], pad, user_emb[:, H:], pad], axis=0)
    xi = jnp.concatenate([item_emb[:, :H], pad, item_emb[:, H:], pad], axis=0)

    outs_u = [user_emb]
    outs_i = [item_emb]
    for w in (W0, W1):
        yu = _spmm(xu, u_col, u_row, u_val)
        yi = _spmm(xi, i_col, i_row, i_val)
        zu, hu0, hu1 = _dense(yu, w)
        zi, hi0, hi1 = _dense(yi, w)
        xu = jnp.concatenate([hu0, hu1], axis=0)
        xi = jnp.concatenate([hi0, hi1], axis=0)
        outs_u.append(zu[:N])
        outs_i.append(zi[:N])

    return jnp.concatenate(outs_u, axis=1), jnp.concatenate(outs_i, axis=1)
